# baseline (speedup 1.0000x reference)
import functools

import jax
import jax.numpy as jnp
import numpy as np
from jax.experimental import pallas as pl
from jax.experimental.pallas import tpu as pltpu
from jax.experimental.shard_map import shard_map
from jax.sharding import Mesh, PartitionSpec as P

LANE = 128


def _round_up(x, m):
    return (x + m - 1) // m * m


def _conv_bn_stats_kernel(x_ref, pscale_ref, pshift_ref, w_ref, b_ref,
                          y_ref, s_ref, ss_ref,
                          xp_ref, b3_ref, *, apply_prologue):
    H, W, Cout = y_ref.shape[1], y_ref.shape[2], y_ref.shape[3]
    Cin = x_ref.shape[3]
    WP = _round_up(W + 2, 8)
    F = (H + 4) * WP
    M = H * WP

    x = x_ref[0]
    if apply_prologue:
        x = jnp.maximum(x * pscale_ref[...] + pshift_ref[...], 0.0)

    xp_ref[0:2 * WP, :] = jnp.zeros((2 * WP, Cin), jnp.float32)
    xp_ref[(H + 2) * WP:F, :] = jnp.zeros((2 * WP, Cin), jnp.float32)
    for i in range(H):
        base = (i + 2) * WP
        xp_ref[base:base + W, :] = x[i]
        xp_ref[base + W:base + WP, :] = jnp.zeros((WP - W, Cin), jnp.float32)

    b3_ref[1:F, 0:Cin] = xp_ref[0:F - 1, :]
    b3_ref[0:F, Cin:2 * Cin] = xp_ref[0:F, :]
    b3_ref[0:F - 1, 2 * Cin:3 * Cin] = xp_ref[1:F, :]

    y = jnp.dot(b3_ref[WP:WP + M, :], w_ref[0],
                preferred_element_type=jnp.float32)
    y = y + jnp.dot(b3_ref[2 * WP:2 * WP + M, :], w_ref[1],
                    preferred_element_type=jnp.float32)
    y = y + jnp.dot(b3_ref[3 * WP:3 * WP + M, :], w_ref[2],
                    preferred_element_type=jnp.float32)
    y = y + b_ref[...]

    yv = y.reshape(H, WP, Cout)[:, 0:W, :]
    y_ref[0] = yv
    yf = yv.reshape(H * W, Cout)
    s_ref[0] = jnp.sum(yf, axis=0, keepdims=True)
    ss_ref[0] = jnp.sum(yf * yf, axis=0, keepdims=True)


def _conv3x3_bn_stats(x, w_mat, b, pre_scale, pre_shift, *, apply_prologue):
    N, H, W, Cin = x.shape
    Cout = w_mat.shape[2]
    WP = _round_up(W + 2, 8)
    _body = functools.partial(_conv_bn_stats_kernel, apply_prologue=apply_prologue)
    flops = 2 * N * H * WP * 9 * Cin * Cout
    bytes_accessed = 4 * (x.size + w_mat.size + N * H * W * Cout)
    return pl.pallas_call(
        _body,
        out_shape=(jax.ShapeDtypeStruct((N, H, W, Cout), jnp.float32),
                   jax.ShapeDtypeStruct((N, 1, Cout), jnp.float32),
                   jax.ShapeDtypeStruct((N, 1, Cout), jnp.float32)),
        grid=(N,),
        in_specs=[
            pl.BlockSpec((1, H, W, Cin), lambda n: (n, 0, 0, 0)),
            pl.BlockSpec((1, Cin), lambda n: (0, 0)),
            pl.BlockSpec((1, Cin), lambda n: (0, 0)),
            pl.BlockSpec((3, 3 * Cin, Cout), lambda n: (0, 0, 0)),
            pl.BlockSpec((1, Cout), lambda n: (0, 0)),
        ],
        out_specs=(
            pl.BlockSpec((1, H, W, Cout), lambda n: (n, 0, 0, 0)),
            pl.BlockSpec((1, 1, Cout), lambda n: (n, 0, 0)),
            pl.BlockSpec((1, 1, Cout), lambda n: (n, 0, 0)),
        ),
        scratch_shapes=[
            pltpu.VMEM(((H + 4) * WP, Cin), jnp.float32),
            pltpu.VMEM(((H + 4) * WP, 3 * Cin), jnp.float32),
        ],
        compiler_params=pltpu.CompilerParams(
            dimension_semantics=("parallel",)),
        cost_estimate=pl.CostEstimate(flops=flops, transcendentals=0,
                                      bytes_accessed=bytes_accessed),
    )(x, pre_scale, pre_shift, w_mat, b)


def _bn_scale_shift(s, ss, count, gamma, beta, eps):
    mean = s / count
    var = jnp.maximum(ss / count - mean * mean, 0.0)
    scale = gamma * jax.lax.rsqrt(var + eps)
    shift = beta - mean * scale
    return scale.reshape(1, -1), shift.reshape(1, -1)


def _prep_w(w, ci, co, cpi, cpo):
    wp = jnp.zeros((3, 3, cpi, cpo), jnp.float32)
    wp = wp.at[:, :, :ci, :co].set(w.astype(jnp.float32))
    return wp.reshape(3, 3 * cpi, cpo)


def _pad_vec(v, cp):
    return jnp.pad(v.astype(jnp.float32), (0, cp - v.shape[0]))


def _double_conv_forward(x_nchw, params, eps=1e-5):
    N, Cin, H, W = x_nchw.shape
    Cout = params["w1"].shape[-1]
    cp_in, cp_out = _round_up(Cin, LANE), _round_up(Cout, LANE)

    w1 = _prep_w(params["w1"], Cin, Cout, cp_in, cp_out)
    w2 = _prep_w(params["w2"], Cout, Cout, cp_out, cp_out)
    b1 = _pad_vec(params["b1"], cp_out).reshape(1, cp_out)
    b2 = _pad_vec(params["b2"], cp_out).reshape(1, cp_out)
    g1, be1 = _pad_vec(params["g1"], cp_out), _pad_vec(params["be1"], cp_out)
    g2, be2 = _pad_vec(params["g2"], cp_out), _pad_vec(params["be2"], cp_out)

    count = float(N * H * W)
    ident = jnp.ones((1, cp_in), jnp.float32)
    zeros = jnp.zeros((1, cp_in), jnp.float32)

    def _fwd_local(x_loc, w1, b1, g1, be1, w2, b2, g2, be2, ident, zeros):
        x = jnp.transpose(x_loc, (0, 2, 3, 1)).astype(jnp.float32)
        if cp_in != Cin:
            x = jnp.pad(x, ((0, 0), (0, 0), (0, 0), (0, cp_in - Cin)))

        y1, s1, ss1 = _conv3x3_bn_stats(x, w1, b1, ident, zeros,
                                        apply_prologue=False)
        s1 = jax.lax.psum(jnp.sum(s1, axis=(0, 1)), "d")
        ss1 = jax.lax.psum(jnp.sum(ss1, axis=(0, 1)), "d")
        sc1, sh1 = _bn_scale_shift(s1, ss1, count, g1, be1, eps)

        y2, s2, ss2 = _conv3x3_bn_stats(y1, w2, b2, sc1, sh1,
                                        apply_prologue=True)
        s2 = jax.lax.psum(jnp.sum(s2, axis=(0, 1)), "d")
        ss2 = jax.lax.psum(jnp.sum(ss2, axis=(0, 1)), "d")
        sc2, sh2 = _bn_scale_shift(s2, ss2, count, g2, be2, eps)

        out = jnp.maximum(
            y2 * sc2.reshape(1, 1, 1, -1) + sh2.reshape(1, 1, 1, -1), 0.0)
        return jnp.transpose(out[..., :Cout], (0, 3, 1, 2))

    devs = [d for d in jax.devices() if d.platform == "tpu"] or jax.devices()
    ndev = 2 if (len(devs) >= 2 and N % 2 == 0) else 1
    mesh = Mesh(np.array(devs[:ndev]), ("d",))
    rep = P()
    fwd = shard_map(
        _fwd_local, mesh=mesh,
        in_specs=(P("d"), rep, rep, rep, rep, rep, rep, rep, rep, rep, rep),
        out_specs=P("d"),
        check_rep=False)
    return fwd(x_nchw, w1, b1, g1, be1, w2, b2, g2, be2, ident, zeros)


_double_conv_forward = jax.jit(_double_conv_forward, static_argnames=())


def kernel(x, w1, b1, g1, be1, w2, b2, g2, be2):
    params = {"w1": w1, "b1": b1, "g1": g1, "be1": be1,
              "w2": w2, "b2": b2, "g2": g2, "be2": be2}
    return _double_conv_forward(x, params)


# speedup vs baseline: 3.2586x; 3.2586x over previous
import functools

import jax
import jax.numpy as jnp
import numpy as np
from jax.experimental import pallas as pl
from jax.experimental.pallas import tpu as pltpu
from jax.experimental.shard_map import shard_map
from jax.sharding import Mesh, PartitionSpec as P

LANE = 128


def _round_up(x, m):
    return (x + m - 1) // m * m


def _conv_bn_stats_kernel(x_ref, pscale_ref, pshift_ref, w_ref, b_ref,
                          y_ref, s_ref, ss_ref,
                          b3_ref, *, apply_prologue):
    H, W, Cout = y_ref.shape[1], y_ref.shape[2], y_ref.shape[3]
    Cin = x_ref.shape[3]
    HW = H * W
    F = _round_up((H + 2) * W + 2, 8)

    x = x_ref[0].astype(jnp.float32)
    if apply_prologue:
        x = jnp.maximum(x * pscale_ref[...] + pshift_ref[...], 0.0)

    b3_ref[0:W, Cin:2 * Cin] = jnp.zeros((W, Cin), jnp.float32)
    b3_ref[W:W + HW, Cin:2 * Cin] = x.reshape(HW, Cin)
    b3_ref[W + HW:F, Cin:2 * Cin] = jnp.zeros((F - W - HW, Cin), jnp.float32)

    mid_lo = b3_ref[0:F - 1, Cin:2 * Cin]
    mid_hi = b3_ref[1:F, Cin:2 * Cin]
    it = jax.lax.broadcasted_iota(jnp.int32, (F - 1, Cin), 0)
    edge = (it % W) == (W - 1)
    b3_ref[1:F, 0:Cin] = jnp.where(edge, 0.0, mid_lo)
    b3_ref[0:1, 0:Cin] = jnp.zeros((1, Cin), jnp.float32)
    b3_ref[0:F - 1, 2 * Cin:3 * Cin] = jnp.where(edge, 0.0, mid_hi)

    y = jnp.dot(b3_ref[0:HW, :], w_ref[0],
                preferred_element_type=jnp.float32)
    y = y + jnp.dot(b3_ref[W:W + HW, :], w_ref[1],
                    preferred_element_type=jnp.float32)
    y = y + jnp.dot(b3_ref[2 * W:2 * W + HW, :], w_ref[2],
                    preferred_element_type=jnp.float32)
    y = y + b_ref[...]

    y_ref[0] = y.reshape(H, W, Cout).astype(jnp.bfloat16)
    s_ref[0] = jnp.sum(y, axis=0, keepdims=True)
    ss_ref[0] = jnp.sum(y * y, axis=0, keepdims=True)


def _conv3x3_bn_stats(x, w_mat, b, pre_scale, pre_shift, *, apply_prologue):
    N, H, W, Cin = x.shape
    Cout = w_mat.shape[2]
    F = _round_up((H + 2) * W + 2, 8)
    _body = functools.partial(_conv_bn_stats_kernel, apply_prologue=apply_prologue)
    flops = 2 * N * H * W * 9 * Cin * Cout
    bytes_accessed = x.size * x.dtype.itemsize + 4 * w_mat.size + 2 * N * H * W * Cout
    return pl.pallas_call(
        _body,
        out_shape=(jax.ShapeDtypeStruct((N, H, W, Cout), jnp.bfloat16),
                   jax.ShapeDtypeStruct((N, 1, Cout), jnp.float32),
                   jax.ShapeDtypeStruct((N, 1, Cout), jnp.float32)),
        grid=(N,),
        in_specs=[
            pl.BlockSpec((1, H, W, Cin), lambda n: (n, 0, 0, 0)),
            pl.BlockSpec((1, Cin), lambda n: (0, 0)),
            pl.BlockSpec((1, Cin), lambda n: (0, 0)),
            pl.BlockSpec((3, 3 * Cin, Cout), lambda n: (0, 0, 0)),
            pl.BlockSpec((1, Cout), lambda n: (0, 0)),
        ],
        out_specs=(
            pl.BlockSpec((1, H, W, Cout), lambda n: (n, 0, 0, 0)),
            pl.BlockSpec((1, 1, Cout), lambda n: (n, 0, 0)),
            pl.BlockSpec((1, 1, Cout), lambda n: (n, 0, 0)),
        ),
        scratch_shapes=[
            pltpu.VMEM((F, 3 * Cin), jnp.float32),
        ],
        compiler_params=pltpu.CompilerParams(
            dimension_semantics=("parallel",)),
        cost_estimate=pl.CostEstimate(flops=flops, transcendentals=0,
                                      bytes_accessed=bytes_accessed),
    )(x, pre_scale, pre_shift, w_mat, b)


def _bn_scale_shift(s, ss, count, gamma, beta, eps):
    mean = s / count
    var = jnp.maximum(ss / count - mean * mean, 0.0)
    scale = gamma * jax.lax.rsqrt(var + eps)
    shift = beta - mean * scale
    return scale.reshape(1, -1), shift.reshape(1, -1)


def _prep_w(w, ci, co, cpi, cpo):
    wp = jnp.zeros((3, 3, cpi, cpo), jnp.float32)
    wp = wp.at[:, :, :ci, :co].set(w.astype(jnp.float32))
    return wp.reshape(3, 3 * cpi, cpo)


def _pad_vec(v, cp):
    return jnp.pad(v.astype(jnp.float32), (0, cp - v.shape[0]))


def _double_conv_forward(x_nchw, params, eps=1e-5):
    N, Cin, H, W = x_nchw.shape
    Cout = params["w1"].shape[-1]
    cp_in, cp_out = _round_up(Cin, LANE), _round_up(Cout, LANE)

    w1 = _prep_w(params["w1"], Cin, Cout, cp_in, cp_out)
    w2 = _prep_w(params["w2"], Cout, Cout, cp_out, cp_out)
    b1 = _pad_vec(params["b1"], cp_out).reshape(1, cp_out)
    b2 = _pad_vec(params["b2"], cp_out).reshape(1, cp_out)
    g1, be1 = _pad_vec(params["g1"], cp_out), _pad_vec(params["be1"], cp_out)
    g2, be2 = _pad_vec(params["g2"], cp_out), _pad_vec(params["be2"], cp_out)

    count = float(N * H * W)
    ident = jnp.ones((1, cp_in), jnp.float32)
    zeros = jnp.zeros((1, cp_in), jnp.float32)

    x = jnp.transpose(x_nchw, (0, 2, 3, 1)).astype(jnp.float32)
    if cp_in != Cin:
        x = jnp.pad(x, ((0, 0), (0, 0), (0, 0), (0, cp_in - Cin)))

    y1, s1, ss1 = _conv3x3_bn_stats(x, w1, b1, ident, zeros,
                                    apply_prologue=False)
    sc1, sh1 = _bn_scale_shift(jnp.sum(s1, axis=(0, 1)),
                               jnp.sum(ss1, axis=(0, 1)), count, g1, be1, eps)

    y2, s2, ss2 = _conv3x3_bn_stats(y1, w2, b2, sc1, sh1,
                                    apply_prologue=True)
    sc2, sh2 = _bn_scale_shift(jnp.sum(s2, axis=(0, 1)),
                               jnp.sum(ss2, axis=(0, 1)), count, g2, be2, eps)

    out = jnp.maximum(
        y2 * sc2.reshape(1, 1, 1, -1) + sh2.reshape(1, 1, 1, -1), 0.0)
    return jnp.transpose(out[..., :Cout], (0, 3, 1, 2))


_double_conv_forward = jax.jit(_double_conv_forward, static_argnames=())


def kernel(x, w1, b1, g1, be1, w2, b2, g2, be2):
    params = {"w1": w1, "b1": b1, "g1": g1, "be1": be1,
              "w2": w2, "b2": b2, "g2": g2, "be2": be2}
    return _double_conv_forward(x, params)


# speedup vs baseline: 3.3300x; 1.0219x over previous
import functools

import jax
import jax.numpy as jnp
import numpy as np
from jax.experimental import pallas as pl
from jax.experimental.pallas import tpu as pltpu
from jax.experimental.shard_map import shard_map
from jax.sharding import Mesh, PartitionSpec as P

LANE = 128


def _round_up(x, m):
    return (x + m - 1) // m * m


def _conv_bn_stats_kernel(x_ref, pscale_ref, pshift_ref, w_ref, b_ref,
                          y_ref, s_ref, ss_ref,
                          b3_ref, *, apply_prologue):
    G, H, W, Cout = y_ref.shape[0], y_ref.shape[1], y_ref.shape[2], y_ref.shape[3]
    Cin = x_ref.shape[3]
    HW = H * W
    F = _round_up((H + 2) * W + 2, 8)

    it = jax.lax.broadcasted_iota(jnp.int32, (F - 1, Cin), 0)
    edge = (it % W) == (W - 1)

    for g in range(G):
        x = x_ref[g].astype(jnp.float32)
        if apply_prologue:
            x = jnp.maximum(x * pscale_ref[...] + pshift_ref[...], 0.0)

        b3_ref[0:W, Cin:2 * Cin] = jnp.zeros((W, Cin), jnp.float32)
        b3_ref[W:W + HW, Cin:2 * Cin] = x.reshape(HW, Cin)
        b3_ref[W + HW:F, Cin:2 * Cin] = jnp.zeros((F - W - HW, Cin), jnp.float32)

        mid_lo = b3_ref[0:F - 1, Cin:2 * Cin]
        mid_hi = b3_ref[1:F, Cin:2 * Cin]
        b3_ref[1:F, 0:Cin] = jnp.where(edge, 0.0, mid_lo)
        b3_ref[0:1, 0:Cin] = jnp.zeros((1, Cin), jnp.float32)
        b3_ref[0:F - 1, 2 * Cin:3 * Cin] = jnp.where(edge, 0.0, mid_hi)

        y = jnp.dot(b3_ref[0:HW, :], w_ref[0],
                    preferred_element_type=jnp.float32)
        y = y + jnp.dot(b3_ref[W:W + HW, :], w_ref[1],
                        preferred_element_type=jnp.float32)
        y = y + jnp.dot(b3_ref[2 * W:2 * W + HW, :], w_ref[2],
                        preferred_element_type=jnp.float32)
        y = y + b_ref[...]

        y_ref[g] = y.reshape(H, W, Cout).astype(jnp.bfloat16)
        s_ref[g] = jnp.sum(y, axis=0, keepdims=True)
        ss_ref[g] = jnp.sum(y * y, axis=0, keepdims=True)


def _conv3x3_bn_stats(x, w_mat, b, pre_scale, pre_shift, *, apply_prologue):
    N, H, W, Cin = x.shape
    Cout = w_mat.shape[2]
    F = _round_up((H + 2) * W + 2, 8)
    G = 4 if N % 4 == 0 else 1
    _body = functools.partial(_conv_bn_stats_kernel, apply_prologue=apply_prologue)
    flops = 2 * N * H * W * 9 * Cin * Cout
    bytes_accessed = x.size * x.dtype.itemsize + 4 * w_mat.size + 2 * N * H * W * Cout
    return pl.pallas_call(
        _body,
        out_shape=(jax.ShapeDtypeStruct((N, H, W, Cout), jnp.bfloat16),
                   jax.ShapeDtypeStruct((N, 1, Cout), jnp.float32),
                   jax.ShapeDtypeStruct((N, 1, Cout), jnp.float32)),
        grid=(N // G,),
        in_specs=[
            pl.BlockSpec((G, H, W, Cin), lambda n: (n, 0, 0, 0)),
            pl.BlockSpec((1, Cin), lambda n: (0, 0)),
            pl.BlockSpec((1, Cin), lambda n: (0, 0)),
            pl.BlockSpec((3, 3 * Cin, Cout), lambda n: (0, 0, 0)),
            pl.BlockSpec((1, Cout), lambda n: (0, 0)),
        ],
        out_specs=(
            pl.BlockSpec((G, H, W, Cout), lambda n: (n, 0, 0, 0)),
            pl.BlockSpec((G, 1, Cout), lambda n: (n, 0, 0)),
            pl.BlockSpec((G, 1, Cout), lambda n: (n, 0, 0)),
        ),
        scratch_shapes=[
            pltpu.VMEM((F, 3 * Cin), jnp.float32),
        ],
        compiler_params=pltpu.CompilerParams(
            dimension_semantics=("parallel",)),
        cost_estimate=pl.CostEstimate(flops=flops, transcendentals=0,
                                      bytes_accessed=bytes_accessed),
    )(x, pre_scale, pre_shift, w_mat, b)


def _bn_scale_shift(s, ss, count, gamma, beta, eps):
    mean = s / count
    var = jnp.maximum(ss / count - mean * mean, 0.0)
    scale = gamma * jax.lax.rsqrt(var + eps)
    shift = beta - mean * scale
    return scale.reshape(1, -1), shift.reshape(1, -1)


def _prep_w(w, ci, co, cpi, cpo):
    wp = jnp.zeros((3, 3, cpi, cpo), jnp.float32)
    wp = wp.at[:, :, :ci, :co].set(w.astype(jnp.float32))
    return wp.reshape(3, 3 * cpi, cpo)


def _pad_vec(v, cp):
    return jnp.pad(v.astype(jnp.float32), (0, cp - v.shape[0]))


def _double_conv_forward(x_nchw, params, eps=1e-5):
    N, Cin, H, W = x_nchw.shape
    Cout = params["w1"].shape[-1]
    cp_in, cp_out = _round_up(Cin, LANE), _round_up(Cout, LANE)

    w1 = _prep_w(params["w1"], Cin, Cout, cp_in, cp_out)
    w2 = _prep_w(params["w2"], Cout, Cout, cp_out, cp_out)
    b1 = _pad_vec(params["b1"], cp_out).reshape(1, cp_out)
    b2 = _pad_vec(params["b2"], cp_out).reshape(1, cp_out)
    g1, be1 = _pad_vec(params["g1"], cp_out), _pad_vec(params["be1"], cp_out)
    g2, be2 = _pad_vec(params["g2"], cp_out), _pad_vec(params["be2"], cp_out)

    count = float(N * H * W)
    ident = jnp.ones((1, cp_in), jnp.float32)
    zeros = jnp.zeros((1, cp_in), jnp.float32)

    x = jnp.transpose(x_nchw, (0, 2, 3, 1)).astype(jnp.float32)
    if cp_in != Cin:
        x = jnp.pad(x, ((0, 0), (0, 0), (0, 0), (0, cp_in - Cin)))

    y1, s1, ss1 = _conv3x3_bn_stats(x, w1, b1, ident, zeros,
                                    apply_prologue=False)
    sc1, sh1 = _bn_scale_shift(jnp.sum(s1, axis=(0, 1)),
                               jnp.sum(ss1, axis=(0, 1)), count, g1, be1, eps)

    y2, s2, ss2 = _conv3x3_bn_stats(y1, w2, b2, sc1, sh1,
                                    apply_prologue=True)
    sc2, sh2 = _bn_scale_shift(jnp.sum(s2, axis=(0, 1)),
                               jnp.sum(ss2, axis=(0, 1)), count, g2, be2, eps)

    out = jnp.maximum(
        y2 * sc2.reshape(1, 1, 1, -1) + sh2.reshape(1, 1, 1, -1), 0.0)
    return jnp.transpose(out[..., :Cout], (0, 3, 1, 2))


_double_conv_forward = jax.jit(_double_conv_forward, static_argnames=())


def kernel(x, w1, b1, g1, be1, w2, b2, g2, be2):
    params = {"w1": w1, "b1": b1, "g1": g1, "be1": be1,
              "w2": w2, "b2": b2, "g2": g2, "be2": be2}
    return _double_conv_forward(x, params)


# speedup vs baseline: 3.3348x; 1.0014x over previous
import functools

import jax
import jax.numpy as jnp
import numpy as np
from jax.experimental import pallas as pl
from jax.experimental.pallas import tpu as pltpu
from jax.experimental.shard_map import shard_map
from jax.sharding import Mesh, PartitionSpec as P

LANE = 128


def _round_up(x, m):
    return (x + m - 1) // m * m


def _conv_bn_stats_kernel(x_ref, pscale_ref, pshift_ref, w_ref, b_ref,
                          y_ref, s_ref, ss_ref,
                          b3_ref, *, apply_prologue):
    G, H, W, Cout = y_ref.shape[0], y_ref.shape[1], y_ref.shape[2], y_ref.shape[3]
    Cin = x_ref.shape[3]
    HW = H * W
    F = _round_up((H + 2) * W + 2, 8)

    it = jax.lax.broadcasted_iota(jnp.int32, (F - 1, Cin), 0)
    edge = (it % W) == (W - 1)

    for g in range(G):
        x = x_ref[g].astype(jnp.float32)
        if apply_prologue:
            x = jnp.maximum(x * pscale_ref[...] + pshift_ref[...], 0.0)

        b3_ref[0:W, Cin:2 * Cin] = jnp.zeros((W, Cin), jnp.float32)
        b3_ref[W:W + HW, Cin:2 * Cin] = x.reshape(HW, Cin)
        b3_ref[W + HW:F, Cin:2 * Cin] = jnp.zeros((F - W - HW, Cin), jnp.float32)

        mid_lo = b3_ref[0:F - 1, Cin:2 * Cin]
        mid_hi = b3_ref[1:F, Cin:2 * Cin]
        b3_ref[1:F, 0:Cin] = jnp.where(edge, 0.0, mid_lo)
        b3_ref[0:1, 0:Cin] = jnp.zeros((1, Cin), jnp.float32)
        b3_ref[0:F - 1, 2 * Cin:3 * Cin] = jnp.where(edge, 0.0, mid_hi)

        y = jnp.dot(b3_ref[0:HW, :], w_ref[0],
                    preferred_element_type=jnp.float32)
        y = y + jnp.dot(b3_ref[W:W + HW, :], w_ref[1],
                        preferred_element_type=jnp.float32)
        y = y + jnp.dot(b3_ref[2 * W:2 * W + HW, :], w_ref[2],
                        preferred_element_type=jnp.float32)
        y = y + b_ref[...]

        y_ref[g] = y.reshape(H, W, Cout).astype(jnp.bfloat16)
        s_ref[g] = jnp.sum(y, axis=0, keepdims=True)
        ss_ref[g] = jnp.sum(y * y, axis=0, keepdims=True)


def _conv3x3_bn_stats(x, w_mat, b, pre_scale, pre_shift, *, apply_prologue):
    N, H, W, Cin = x.shape
    Cout = w_mat.shape[2]
    F = _round_up((H + 2) * W + 2, 8)
    G = 2 if N % 2 == 0 else 1
    _body = functools.partial(_conv_bn_stats_kernel, apply_prologue=apply_prologue)
    flops = 2 * N * H * W * 9 * Cin * Cout
    bytes_accessed = x.size * x.dtype.itemsize + 4 * w_mat.size + 2 * N * H * W * Cout
    return pl.pallas_call(
        _body,
        out_shape=(jax.ShapeDtypeStruct((N, H, W, Cout), jnp.bfloat16),
                   jax.ShapeDtypeStruct((N, 1, Cout), jnp.float32),
                   jax.ShapeDtypeStruct((N, 1, Cout), jnp.float32)),
        grid=(N // G,),
        in_specs=[
            pl.BlockSpec((G, H, W, Cin), lambda n: (n, 0, 0, 0)),
            pl.BlockSpec((1, Cin), lambda n: (0, 0)),
            pl.BlockSpec((1, Cin), lambda n: (0, 0)),
            pl.BlockSpec((3, 3 * Cin, Cout), lambda n: (0, 0, 0)),
            pl.BlockSpec((1, Cout), lambda n: (0, 0)),
        ],
        out_specs=(
            pl.BlockSpec((G, H, W, Cout), lambda n: (n, 0, 0, 0)),
            pl.BlockSpec((G, 1, Cout), lambda n: (n, 0, 0)),
            pl.BlockSpec((G, 1, Cout), lambda n: (n, 0, 0)),
        ),
        scratch_shapes=[
            pltpu.VMEM((F, 3 * Cin), jnp.float32),
        ],
        compiler_params=pltpu.CompilerParams(
            dimension_semantics=("parallel",)),
        cost_estimate=pl.CostEstimate(flops=flops, transcendentals=0,
                                      bytes_accessed=bytes_accessed),
    )(x, pre_scale, pre_shift, w_mat, b)


def _bn_scale_shift(s, ss, count, gamma, beta, eps):
    mean = s / count
    var = jnp.maximum(ss / count - mean * mean, 0.0)
    scale = gamma * jax.lax.rsqrt(var + eps)
    shift = beta - mean * scale
    return scale.reshape(1, -1), shift.reshape(1, -1)


def _prep_w(w, ci, co, cpi, cpo):
    wp = jnp.zeros((3, 3, cpi, cpo), jnp.float32)
    wp = wp.at[:, :, :ci, :co].set(w.astype(jnp.float32))
    return wp.reshape(3, 3 * cpi, cpo)


def _pad_vec(v, cp):
    return jnp.pad(v.astype(jnp.float32), (0, cp - v.shape[0]))


def _double_conv_forward(x_nchw, params, eps=1e-5):
    N, Cin, H, W = x_nchw.shape
    Cout = params["w1"].shape[-1]
    cp_in, cp_out = _round_up(Cin, LANE), _round_up(Cout, LANE)

    w1 = _prep_w(params["w1"], Cin, Cout, cp_in, cp_out)
    w2 = _prep_w(params["w2"], Cout, Cout, cp_out, cp_out)
    b1 = _pad_vec(params["b1"], cp_out).reshape(1, cp_out)
    b2 = _pad_vec(params["b2"], cp_out).reshape(1, cp_out)
    g1, be1 = _pad_vec(params["g1"], cp_out), _pad_vec(params["be1"], cp_out)
    g2, be2 = _pad_vec(params["g2"], cp_out), _pad_vec(params["be2"], cp_out)

    count = float(N * H * W)
    ident = jnp.ones((1, cp_in), jnp.float32)
    zeros = jnp.zeros((1, cp_in), jnp.float32)

    x = jnp.transpose(x_nchw, (0, 2, 3, 1)).astype(jnp.float32)
    if cp_in != Cin:
        x = jnp.pad(x, ((0, 0), (0, 0), (0, 0), (0, cp_in - Cin)))

    y1, s1, ss1 = _conv3x3_bn_stats(x, w1, b1, ident, zeros,
                                    apply_prologue=False)
    sc1, sh1 = _bn_scale_shift(jnp.sum(s1, axis=(0, 1)),
                               jnp.sum(ss1, axis=(0, 1)), count, g1, be1, eps)

    y2, s2, ss2 = _conv3x3_bn_stats(y1, w2, b2, sc1, sh1,
                                    apply_prologue=True)
    sc2, sh2 = _bn_scale_shift(jnp.sum(s2, axis=(0, 1)),
                               jnp.sum(ss2, axis=(0, 1)), count, g2, be2, eps)

    out = jnp.maximum(
        y2 * sc2.reshape(1, 1, 1, -1) + sh2.reshape(1, 1, 1, -1), 0.0)
    return jnp.transpose(out[..., :Cout], (0, 3, 1, 2))


_double_conv_forward = jax.jit(_double_conv_forward, static_argnames=())


def kernel(x, w1, b1, g1, be1, w2, b2, g2, be2):
    params = {"w1": w1, "b1": b1, "g1": g1, "be1": be1,
              "w2": w2, "b2": b2, "g2": g2, "be2": be2}
    return _double_conv_forward(x, params)


# speedup vs baseline: 3.3358x; 1.0003x over previous
import functools

import jax
import jax.numpy as jnp
import numpy as np
from jax.experimental import pallas as pl
from jax.experimental.pallas import tpu as pltpu
from jax.experimental.shard_map import shard_map
from jax.sharding import Mesh, PartitionSpec as P

LANE = 128


def _round_up(x, m):
    return (x + m - 1) // m * m


def _conv_bn_stats_kernel(x_ref, pscale_ref, pshift_ref, w_ref, b_ref,
                          y_ref, s_ref, ss_ref,
                          b3_ref, patches_ref=None, *, apply_prologue):
    G, H, W, Cout = y_ref.shape[0], y_ref.shape[1], y_ref.shape[2], y_ref.shape[3]
    Cin = x_ref.shape[3]
    HW = H * W
    F = _round_up((H + 2) * W + 2, 8)

    it = jax.lax.broadcasted_iota(jnp.int32, (F - 1, Cin), 0)
    edge = (it % W) == (W - 1)

    for g in range(G):
        x = x_ref[g].astype(jnp.float32)
        if apply_prologue:
            x = jnp.maximum(x * pscale_ref[...] + pshift_ref[...], 0.0)

        b3_ref[0:W, Cin:2 * Cin] = jnp.zeros((W, Cin), jnp.float32)
        b3_ref[W:W + HW, Cin:2 * Cin] = x.reshape(HW, Cin)
        b3_ref[W + HW:F, Cin:2 * Cin] = jnp.zeros((F - W - HW, Cin), jnp.float32)

        mid_lo = b3_ref[0:F - 1, Cin:2 * Cin]
        mid_hi = b3_ref[1:F, Cin:2 * Cin]
        b3_ref[1:F, 0:Cin] = jnp.where(edge, 0.0, mid_lo)
        b3_ref[0:1, 0:Cin] = jnp.zeros((1, Cin), jnp.float32)
        b3_ref[0:F - 1, 2 * Cin:3 * Cin] = jnp.where(edge, 0.0, mid_hi)

        if patches_ref is not None:
            for ky in range(3):
                patches_ref[:, ky * 3 * Cin:(ky + 1) * 3 * Cin] = (
                    b3_ref[ky * W:ky * W + HW, :])
            y = jnp.dot(patches_ref[...], w_ref[...],
                        preferred_element_type=jnp.float32)
        else:
            y = jnp.dot(b3_ref[0:HW, :], w_ref[0:3 * Cin],
                        preferred_element_type=jnp.float32)
            y = y + jnp.dot(b3_ref[W:W + HW, :], w_ref[3 * Cin:6 * Cin],
                            preferred_element_type=jnp.float32)
            y = y + jnp.dot(b3_ref[2 * W:2 * W + HW, :], w_ref[6 * Cin:9 * Cin],
                            preferred_element_type=jnp.float32)
        y = y + b_ref[...]

        y_ref[g] = y.reshape(H, W, Cout).astype(jnp.bfloat16)
        s_ref[g] = jnp.sum(y, axis=0, keepdims=True)
        ss_ref[g] = jnp.sum(y * y, axis=0, keepdims=True)


def _conv3x3_bn_stats(x, w_mat, b, pre_scale, pre_shift, *, apply_prologue):
    N, H, W, Cin = x.shape
    Cout = w_mat.shape[1]
    F = _round_up((H + 2) * W + 2, 8)
    G = 2 if N % 2 == 0 else 1
    scratch = [pltpu.VMEM((F, 3 * Cin), jnp.float32)]
    _body = functools.partial(_conv_bn_stats_kernel, apply_prologue=apply_prologue)
    flops = 2 * N * H * W * 9 * Cin * Cout
    bytes_accessed = x.size * x.dtype.itemsize + 4 * w_mat.size + 2 * N * H * W * Cout
    return pl.pallas_call(
        _body,
        out_shape=(jax.ShapeDtypeStruct((N, H, W, Cout), jnp.bfloat16),
                   jax.ShapeDtypeStruct((N, 1, Cout), jnp.float32),
                   jax.ShapeDtypeStruct((N, 1, Cout), jnp.float32)),
        grid=(N // G,),
        in_specs=[
            pl.BlockSpec((G, H, W, Cin), lambda n: (n, 0, 0, 0)),
            pl.BlockSpec((1, Cin), lambda n: (0, 0)),
            pl.BlockSpec((1, Cin), lambda n: (0, 0)),
            pl.BlockSpec((9 * Cin, Cout), lambda n: (0, 0)),
            pl.BlockSpec((1, Cout), lambda n: (0, 0)),
        ],
        out_specs=(
            pl.BlockSpec((G, H, W, Cout), lambda n: (n, 0, 0, 0)),
            pl.BlockSpec((G, 1, Cout), lambda n: (n, 0, 0)),
            pl.BlockSpec((G, 1, Cout), lambda n: (n, 0, 0)),
        ),
        scratch_shapes=scratch,
        compiler_params=pltpu.CompilerParams(
            dimension_semantics=("arbitrary",)),
        cost_estimate=pl.CostEstimate(flops=flops, transcendentals=0,
                                      bytes_accessed=bytes_accessed),
    )(x, pre_scale, pre_shift, w_mat, b)


def _bn_scale_shift(s, ss, count, gamma, beta, eps):
    mean = s / count
    var = jnp.maximum(ss / count - mean * mean, 0.0)
    scale = gamma * jax.lax.rsqrt(var + eps)
    shift = beta - mean * scale
    return scale.reshape(1, -1), shift.reshape(1, -1)


def _prep_w(w, ci, co, cpi, cpo):
    wp = jnp.zeros((3, 3, cpi, cpo), jnp.float32)
    wp = wp.at[:, :, :ci, :co].set(w.astype(jnp.float32))
    return wp.reshape(9 * cpi, cpo)


def _pad_vec(v, cp):
    return jnp.pad(v.astype(jnp.float32), (0, cp - v.shape[0]))


def _double_conv_forward(x_nchw, params, eps=1e-5):
    N, Cin, H, W = x_nchw.shape
    Cout = params["w1"].shape[-1]
    cp_in, cp_out = _round_up(Cin, LANE), _round_up(Cout, LANE)

    w1 = _prep_w(params["w1"], Cin, Cout, cp_in, cp_out)
    w2 = _prep_w(params["w2"], Cout, Cout, cp_out, cp_out)
    b1 = _pad_vec(params["b1"], cp_out).reshape(1, cp_out)
    b2 = _pad_vec(params["b2"], cp_out).reshape(1, cp_out)
    g1, be1 = _pad_vec(params["g1"], cp_out), _pad_vec(params["be1"], cp_out)
    g2, be2 = _pad_vec(params["g2"], cp_out), _pad_vec(params["be2"], cp_out)

    count = float(N * H * W)
    ident = jnp.ones((1, cp_in), jnp.float32)
    zeros = jnp.zeros((1, cp_in), jnp.float32)

    x = jnp.transpose(x_nchw, (0, 2, 3, 1)).astype(jnp.float32)
    if cp_in != Cin:
        x = jnp.pad(x, ((0, 0), (0, 0), (0, 0), (0, cp_in - Cin)))

    y1, s1, ss1 = _conv3x3_bn_stats(x, w1, b1, ident, zeros,
                                    apply_prologue=False)
    sc1, sh1 = _bn_scale_shift(jnp.sum(s1, axis=(0, 1)),
                               jnp.sum(ss1, axis=(0, 1)), count, g1, be1, eps)

    y2, s2, ss2 = _conv3x3_bn_stats(y1, w2, b2, sc1, sh1,
                                    apply_prologue=True)
    sc2, sh2 = _bn_scale_shift(jnp.sum(s2, axis=(0, 1)),
                               jnp.sum(ss2, axis=(0, 1)), count, g2, be2, eps)

    out = jnp.maximum(
        y2 * sc2.reshape(1, 1, 1, -1) + sh2.reshape(1, 1, 1, -1), 0.0)
    return jnp.transpose(out[..., :Cout], (0, 3, 1, 2))


_double_conv_forward = jax.jit(_double_conv_forward, static_argnames=())


def kernel(x, w1, b1, g1, be1, w2, b2, g2, be2):
    params = {"w1": w1, "b1": b1, "g1": g1, "be1": be1,
              "w2": w2, "b2": b2, "g2": g2, "be2": be2}
    return _double_conv_forward(x, params)
